# baseline (speedup 1.0000x reference)
import functools

import jax
import jax.numpy as jnp
from jax import lax
from jax.experimental import pallas as pl
from jax.experimental.pallas import tpu as pltpu
from jax.experimental.pallas import tpu_sc as plsc

T = 4096
N_SPANS = 4096
D_HEAD = 1024
D_CTX = 1024
F = 128
ZCOLS = 128
DC = D_HEAD + ZCOLS
D_OUT = D_CTX + D_CTX + F + D_HEAD

BLK = 256
L = 16
NC, NS = 2, 16
NW = NC * NS
SPANS_PER_W = N_SPANS // NW
CHUNK = 8
N_CHUNKS = SPANS_PER_W // CHUNK


def _prefix_body(ctx_ref, head_ref, w_ref, b_ref, out_ref, carry_ref):
    i = pl.program_id(0)

    @pl.when(i == 0)
    def _():
        carry_ref[...] = jnp.zeros_like(carry_ref)

    ctx = ctx_ref[...]
    head = head_ref[...]
    w = w_ref[...]
    b = b_ref[0, 0]
    s = jnp.sum(ctx * w, axis=1, keepdims=True) + b
    e = jnp.exp(s)
    gfull = jnp.concatenate(
        [e * head, jnp.broadcast_to(e, (BLK, ZCOLS))], axis=1)
    r = lax.broadcasted_iota(jnp.int32, (BLK, BLK), 0)
    c = lax.broadcasted_iota(jnp.int32, (BLK, BLK), 1)
    strict_l = (r > c).astype(jnp.float32)
    ex = jnp.dot(strict_l, gfull, preferred_element_type=jnp.float32)
    out_ref[...] = ex + carry_ref[...]
    carry_ref[...] = carry_ref[...] + jnp.sum(gfull, axis=0, keepdims=True)


def _prefix_table(context_outputs, head_emb, attn_w, attn_b):
    return pl.pallas_call(
        _prefix_body,
        grid=(T // BLK,),
        in_specs=[
            pl.BlockSpec((BLK, D_CTX), lambda i: (i, 0)),
            pl.BlockSpec((BLK, D_HEAD), lambda i: (i, 0)),
            pl.BlockSpec((1, D_CTX), lambda i: (0, 0)),
            pl.BlockSpec((1, 1), lambda i: (0, 0)),
        ],
        out_specs=pl.BlockSpec((BLK, DC), lambda i: (i, 0)),
        out_shape=jax.ShapeDtypeStruct((T, DC), jnp.float32),
        scratch_shapes=[pltpu.VMEM((1, DC), jnp.float32)],
    )(context_outputs, head_emb, attn_w, attn_b)


def _worker_base():
    wid = lax.axis_index("s") * NC + lax.axis_index("c")
    return wid * SPANS_PER_W


def _sc_ctx_body(ctx_hbm, swe_hbm, st_hbm, en_hbm, wi_hbm, out_hbm,
                 st_all, en_all, wi_all, we_all,
                 cxs0, cxs1, cxe0, cxe1, gsem0, gsem1, osem0, osem1):
    cxs = (cxs0, cxs1)
    cxe = (cxe0, cxe1)
    gsem = (gsem0, gsem1)
    osem = (osem0, osem1)
    base = _worker_base()

    pltpu.sync_copy(st_hbm.at[pl.ds(base, SPANS_PER_W)], st_all)
    pltpu.sync_copy(en_hbm.at[pl.ds(base, SPANS_PER_W)], en_all)
    pltpu.sync_copy(wi_hbm.at[pl.ds(base, SPANS_PER_W)], wi_all)
    pltpu.async_copy(swe_hbm.at[wi_all], we_all, gsem0).wait()

    def fire_g(c, b):
        sl = pl.ds(c * CHUNK_A, CHUNK_A)
        pltpu.async_copy(ctx_hbm.at[st_all.at[sl]], cxs[b], gsem[b])
        pltpu.async_copy(ctx_hbm.at[en_all.at[sl]], cxe[b], gsem[b])

    def wait_g(b):
        pltpu.make_async_copy(ctx_hbm.at[pl.ds(0, CHUNK_A)], cxs[b],
                              gsem[b]).wait()
        pltpu.make_async_copy(ctx_hbm.at[pl.ds(0, CHUNK_A)], cxe[b],
                              gsem[b]).wait()

    def fire_out(c, b):
        rows = pl.ds(base + c * CHUNK_A, CHUNK_A)
        wsl = pl.ds(c * CHUNK_A, CHUNK_A)
        pltpu.async_copy(cxs[b], out_hbm.at[rows, pl.ds(0, D_CTX)], osem[b])
        pltpu.async_copy(cxe[b], out_hbm.at[rows, pl.ds(D_CTX, D_CTX)],
                         osem[b])
        pltpu.async_copy(we_all.at[wsl], out_hbm.at[rows, pl.ds(2 * D_CTX, F)],
                         osem[b])

    def wait_out(b):
        rows = pl.ds(base, CHUNK_A)
        pltpu.make_async_copy(cxs[b], out_hbm.at[rows, pl.ds(0, D_CTX)],
                              osem[b]).wait()
        pltpu.make_async_copy(cxe[b], out_hbm.at[rows, pl.ds(D_CTX, D_CTX)],
                              osem[b]).wait()
        pltpu.make_async_copy(we_all.at[pl.ds(0, CHUNK_A)],
                              out_hbm.at[rows, pl.ds(2 * D_CTX, F)],
                              osem[b]).wait()

    fire_g(0, 0)

    def outer(k, _):
        for b in range(2):
            c = 2 * k + b
            wait_g(b)
            fire_out(c, b)

            @pl.when(c < N_CHUNKS_A - 1)
            def _():
                @pl.when(c > 0)
                def _():
                    wait_out(1 - b)

                fire_g(c + 1, 1 - b)

        return 0

    lax.fori_loop(0, N_CHUNKS_A // 2, outer, 0)
    wait_out(0)
    wait_out(1)


def _sc_head_body(cz_hbm, st_hbm, enp_hbm, out_hbm,
                  st_all, enp_all, czs0, czs1, cze0, cze1, sh0, sh1,
                  rcp_buf, gsem0, gsem1, osem0, osem1):
    czs = (czs0, czs1)
    cze = (cze0, cze1)
    sh = (sh0, sh1)
    gsem = (gsem0, gsem1)
    osem = (osem0, osem1)
    base = _worker_base()

    pltpu.sync_copy(st_hbm.at[pl.ds(base, SPANS_PER_W)], st_all)
    pltpu.sync_copy(enp_hbm.at[pl.ds(base, SPANS_PER_W)], enp_all)

    def fire_g(c, b):
        sl = pl.ds(c * CHUNK_B, CHUNK_B)
        pltpu.async_copy(cz_hbm.at[st_all.at[sl]], czs[b], gsem[b])
        pltpu.async_copy(cz_hbm.at[enp_all.at[sl]], cze[b], gsem[b])

    def wait_g(b):
        pltpu.make_async_copy(cz_hbm.at[pl.ds(0, CHUNK_B)], czs[b],
                              gsem[b]).wait()
        pltpu.make_async_copy(cz_hbm.at[pl.ds(0, CHUNK_B)], cze[b],
                              gsem[b]).wait()

    def fire_out(c, b):
        rows = pl.ds(base + c * CHUNK_B, CHUNK_B)
        pltpu.async_copy(sh[b], out_hbm.at[rows, pl.ds(2 * D_CTX + F, D_HEAD)],
                         osem[b])

    def wait_out(b):
        rows = pl.ds(base, CHUNK_B)
        pltpu.make_async_copy(sh[b], out_hbm.at[rows, pl.ds(2 * D_CTX + F,
                                                            D_HEAD)],
                              osem[b]).wait()

    def compute(b):
        @plsc.parallel_loop(0, CHUNK_B)
        def _(i):
            den = cze[b][i, pl.ds(D_HEAD, L)] - czs[b][i, pl.ds(D_HEAD, L)]
            rcp_buf[pl.ds(i * L, L)] = 1.0 / den

        def span_body(i, _):
            rcp = rcp_buf[pl.ds(i * L, L)]

            @plsc.parallel_loop(0, D_HEAD, step=L, unroll=8)
            def _(t):
                vsl = pl.ds(t, L)
                sh[b][i, vsl] = (cze[b][i, vsl] - czs[b][i, vsl]) * rcp

            return 0

        lax.fori_loop(0, CHUNK_B, span_body, 0)

    fire_g(0, 0)

    def outer(k, _):
        for b in range(2):
            c = 2 * k + b
            wait_g(b)

            @pl.when(c < N_CHUNKS_B - 1)
            def _():
                fire_g(c + 1, 1 - b)

            @pl.when(c > 1)
            def _():
                wait_out(b)

            fire_out(c, b)

        return 0

    lax.fori_loop(0, N_CHUNKS_B // 2, outer, 0)
    wait_out(0)
    wait_out(1)


CHUNK_A = 16
N_CHUNKS_A = SPANS_PER_W // CHUNK_A
CHUNK_B = 16
N_CHUNKS_B = SPANS_PER_W // CHUNK_B

_SC_MESH = dict(core_axis_name="c", subcore_axis_name="s")


@functools.cache
def _sc_ctx():
    return pl.kernel(
        _sc_ctx_body,
        mesh=plsc.VectorSubcoreMesh(**_SC_MESH),
        out_type=(),
        scratch_types=[
            pltpu.VMEM((SPANS_PER_W,), jnp.int32),
            pltpu.VMEM((SPANS_PER_W,), jnp.int32),
            pltpu.VMEM((SPANS_PER_W,), jnp.int32),
            pltpu.VMEM((SPANS_PER_W, F), jnp.float32),
            pltpu.VMEM((CHUNK_A, D_CTX), jnp.float32),
            pltpu.VMEM((CHUNK_A, D_CTX), jnp.float32),
            pltpu.VMEM((CHUNK_A, D_CTX), jnp.float32),
            pltpu.VMEM((CHUNK_A, D_CTX), jnp.float32),
            pltpu.SemaphoreType.DMA,
            pltpu.SemaphoreType.DMA,
            pltpu.SemaphoreType.DMA,
            pltpu.SemaphoreType.DMA,
        ],
    )


@functools.cache
def _sc_head():
    return pl.kernel(
        _sc_head_body,
        mesh=plsc.VectorSubcoreMesh(**_SC_MESH),
        out_type=(),
        scratch_types=[
            pltpu.VMEM((SPANS_PER_W,), jnp.int32),
            pltpu.VMEM((SPANS_PER_W,), jnp.int32),
            pltpu.VMEM((CHUNK_B, DC), jnp.float32),
            pltpu.VMEM((CHUNK_B, DC), jnp.float32),
            pltpu.VMEM((CHUNK_B, DC), jnp.float32),
            pltpu.VMEM((CHUNK_B, DC), jnp.float32),
            pltpu.VMEM((CHUNK_B, D_HEAD), jnp.float32),
            pltpu.VMEM((CHUNK_B, D_HEAD), jnp.float32),
            pltpu.VMEM((CHUNK_B * L,), jnp.float32),
            pltpu.SemaphoreType.DMA,
            pltpu.SemaphoreType.DMA,
            pltpu.SemaphoreType.DMA,
            pltpu.SemaphoreType.DMA,
        ],
    )


def kernel(head_emb, text_lens, context_outputs, span_starts, span_ends,
           is_training, span_width_embeddings, attn_W, attn_b):
    del text_lens, is_training
    starts = span_starts.astype(jnp.int32)
    ends = span_ends.astype(jnp.int32)
    ends_p1 = ends + 1
    widx = ends - starts

    out_ref = jax.new_ref(jax.lax.empty((N_SPANS, D_OUT), jnp.float32))
    _sc_ctx()(context_outputs, span_width_embeddings, starts, ends, widx,
              out_ref)
    cz = _prefix_table(context_outputs, head_emb, attn_W,
                       attn_b.reshape(1, 1).astype(jnp.float32))
    _sc_head()(cz, starts, ends_p1, out_ref)
    return out_ref[...]


# speedup vs baseline: 1.0044x; 1.0044x over previous
import functools

import jax
import jax.numpy as jnp
from jax import lax
from jax.experimental import pallas as pl
from jax.experimental.pallas import tpu as pltpu
from jax.experimental.pallas import tpu_sc as plsc

T = 4096
N_SPANS = 4096
D_HEAD = 1024
D_CTX = 1024
F = 128
ZCOLS = 128
DC = D_HEAD + ZCOLS
D_OUT = D_CTX + D_CTX + F + D_HEAD

BLK = 256
L = 16
NC, NS = 2, 16
NW = NC * NS
SPANS_PER_W = N_SPANS // NW
CHUNK = 8
N_CHUNKS = SPANS_PER_W // CHUNK


def _prefix_body(ctx_ref, head_ref, w_ref, b_ref, out_ref, carry_ref):
    i = pl.program_id(0)

    @pl.when(i == 0)
    def _():
        carry_ref[...] = jnp.zeros_like(carry_ref)

    ctx = ctx_ref[...]
    head = head_ref[...]
    w = w_ref[...]
    b = b_ref[0, 0]
    s = jnp.sum(ctx * w, axis=1, keepdims=True) + b
    e = jnp.exp(s)
    gfull = jnp.concatenate(
        [e * head, jnp.broadcast_to(e, (BLK, ZCOLS))], axis=1)
    r = lax.broadcasted_iota(jnp.int32, (BLK, BLK), 0)
    c = lax.broadcasted_iota(jnp.int32, (BLK, BLK), 1)
    strict_l = (r > c).astype(jnp.float32)
    ex = jnp.dot(strict_l, gfull, preferred_element_type=jnp.float32)
    out_ref[...] = ex + carry_ref[...]
    carry_ref[...] = carry_ref[...] + jnp.sum(gfull, axis=0, keepdims=True)


def _prefix_table(context_outputs, head_emb, attn_w, attn_b):
    return pl.pallas_call(
        _prefix_body,
        grid=(T // BLK,),
        in_specs=[
            pl.BlockSpec((BLK, D_CTX), lambda i: (i, 0)),
            pl.BlockSpec((BLK, D_HEAD), lambda i: (i, 0)),
            pl.BlockSpec((1, D_CTX), lambda i: (0, 0)),
            pl.BlockSpec((1, 1), lambda i: (0, 0)),
        ],
        out_specs=pl.BlockSpec((BLK, DC), lambda i: (i, 0)),
        out_shape=jax.ShapeDtypeStruct((T, DC), jnp.float32),
        scratch_shapes=[pltpu.VMEM((1, DC), jnp.float32)],
    )(context_outputs, head_emb, attn_w, attn_b)


def _worker_base():
    wid = lax.axis_index("s") * NC + lax.axis_index("c")
    return wid * SPANS_PER_W


def _sc_ctx_body(ctx_hbm, swe_hbm, st_hbm, en_hbm, out_hbm,
                 st_all, en_all, wi_all, we_all,
                 cxs0, cxs1, cxe0, cxe1, gsem0, gsem1, osem0, osem1, wsem):
    cxs = (cxs0, cxs1)
    cxe = (cxe0, cxe1)
    gsem = (gsem0, gsem1)
    osem = (osem0, osem1)
    base = _worker_base()

    pltpu.sync_copy(st_hbm.at[pl.ds(base, SPANS_PER_W)], st_all)
    pltpu.sync_copy(en_hbm.at[pl.ds(base, SPANS_PER_W)], en_all)

    @plsc.parallel_loop(0, SPANS_PER_W, step=L)
    def _(t):
        sl = pl.ds(t, L)
        wi_all[sl] = en_all[sl] - st_all[sl]

    pltpu.async_copy(swe_hbm.at[wi_all], we_all, gsem0).wait()

    def fire_g(c, b):
        sl = pl.ds(c * CHUNK_A, CHUNK_A)
        pltpu.async_copy(ctx_hbm.at[st_all.at[sl]], cxs[b], gsem[b])
        pltpu.async_copy(ctx_hbm.at[en_all.at[sl]], cxe[b], gsem[b])

    def wait_g(b):
        pltpu.make_async_copy(ctx_hbm.at[pl.ds(0, CHUNK_A)], cxs[b],
                              gsem[b]).wait()
        pltpu.make_async_copy(ctx_hbm.at[pl.ds(0, CHUNK_A)], cxe[b],
                              gsem[b]).wait()

    def fire_out(c, b):
        rows = pl.ds(base + c * CHUNK_A, CHUNK_A)
        pltpu.async_copy(cxs[b], out_hbm.at[rows, pl.ds(0, D_CTX)], osem[b])
        pltpu.async_copy(cxe[b], out_hbm.at[rows, pl.ds(D_CTX, D_CTX)],
                         osem[b])

    def wait_out(b):
        rows = pl.ds(base, CHUNK_A)
        pltpu.make_async_copy(cxs[b], out_hbm.at[rows, pl.ds(0, D_CTX)],
                              osem[b]).wait()
        pltpu.make_async_copy(cxe[b], out_hbm.at[rows, pl.ds(D_CTX, D_CTX)],
                              osem[b]).wait()

    we_out = out_hbm.at[pl.ds(base, SPANS_PER_W), pl.ds(2 * D_CTX, F)]
    pltpu.async_copy(we_all, we_out, wsem)
    fire_g(0, 0)

    def outer(k, _):
        for b in range(2):
            c = 2 * k + b
            wait_g(b)
            fire_out(c, b)

            @pl.when(c < N_CHUNKS_A - 1)
            def _():
                @pl.when(c > 0)
                def _():
                    wait_out(1 - b)

                fire_g(c + 1, 1 - b)

        return 0

    lax.fori_loop(0, N_CHUNKS_A // 2, outer, 0)
    wait_out(0)
    wait_out(1)
    pltpu.make_async_copy(we_all, we_out, wsem).wait()


def _sc_head_body(cz_hbm, st_hbm, en_hbm, out_hbm,
                  st_all, enp_all, czs0, czs1, cze0, cze1, sh0, sh1,
                  rcp_buf, gsem0, gsem1, osem0, osem1):
    czs = (czs0, czs1)
    cze = (cze0, cze1)
    sh = (sh0, sh1)
    gsem = (gsem0, gsem1)
    osem = (osem0, osem1)
    base = _worker_base()

    pltpu.sync_copy(st_hbm.at[pl.ds(base, SPANS_PER_W)], st_all)
    pltpu.sync_copy(en_hbm.at[pl.ds(base, SPANS_PER_W)], enp_all)

    @plsc.parallel_loop(0, SPANS_PER_W, step=L)
    def _(t):
        sl = pl.ds(t, L)
        enp_all[sl] = enp_all[sl] + 1

    def fire_g(c, b):
        sl = pl.ds(c * CHUNK_B, CHUNK_B)
        pltpu.async_copy(cz_hbm.at[st_all.at[sl]], czs[b], gsem[b])
        pltpu.async_copy(cz_hbm.at[enp_all.at[sl]], cze[b], gsem[b])

    def wait_g(b):
        pltpu.make_async_copy(cz_hbm.at[pl.ds(0, CHUNK_B)], czs[b],
                              gsem[b]).wait()
        pltpu.make_async_copy(cz_hbm.at[pl.ds(0, CHUNK_B)], cze[b],
                              gsem[b]).wait()

    def fire_out(c, b):
        rows = pl.ds(base + c * CHUNK_B, CHUNK_B)
        pltpu.async_copy(sh[b], out_hbm.at[rows, pl.ds(2 * D_CTX + F, D_HEAD)],
                         osem[b])

    def wait_out(b):
        rows = pl.ds(base, CHUNK_B)
        pltpu.make_async_copy(sh[b], out_hbm.at[rows, pl.ds(2 * D_CTX + F,
                                                            D_HEAD)],
                              osem[b]).wait()

    def compute(b):
        @plsc.parallel_loop(0, CHUNK_B)
        def _(i):
            den = cze[b][i, pl.ds(D_HEAD, L)] - czs[b][i, pl.ds(D_HEAD, L)]
            rcp_buf[pl.ds(i * L, L)] = 1.0 / den

        def span_body(i, _):
            rcp = rcp_buf[pl.ds(i * L, L)]

            @plsc.parallel_loop(0, D_HEAD, step=L, unroll=8)
            def _(t):
                vsl = pl.ds(t, L)
                sh[b][i, vsl] = (cze[b][i, vsl] - czs[b][i, vsl]) * rcp

            return 0

        lax.fori_loop(0, CHUNK_B, span_body, 0)

    fire_g(0, 0)

    def outer(k, _):
        for b in range(2):
            c = 2 * k + b
            wait_g(b)

            @pl.when(c < N_CHUNKS_B - 1)
            def _():
                fire_g(c + 1, 1 - b)

            @pl.when(c > 1)
            def _():
                wait_out(b)

            compute(b)
            fire_out(c, b)

        return 0

    lax.fori_loop(0, N_CHUNKS_B // 2, outer, 0)
    wait_out(0)
    wait_out(1)


CHUNK_A = 16
N_CHUNKS_A = SPANS_PER_W // CHUNK_A
CHUNK_B = 16
N_CHUNKS_B = SPANS_PER_W // CHUNK_B

_SC_MESH = dict(core_axis_name="c", subcore_axis_name="s")


@functools.cache
def _sc_ctx():
    return pl.kernel(
        _sc_ctx_body,
        mesh=plsc.VectorSubcoreMesh(**_SC_MESH),
        out_type=(),
        scratch_types=[
            pltpu.VMEM((SPANS_PER_W,), jnp.int32),
            pltpu.VMEM((SPANS_PER_W,), jnp.int32),
            pltpu.VMEM((SPANS_PER_W,), jnp.int32),
            pltpu.VMEM((SPANS_PER_W, F), jnp.float32),
            pltpu.VMEM((CHUNK_A, D_CTX), jnp.float32),
            pltpu.VMEM((CHUNK_A, D_CTX), jnp.float32),
            pltpu.VMEM((CHUNK_A, D_CTX), jnp.float32),
            pltpu.VMEM((CHUNK_A, D_CTX), jnp.float32),
            pltpu.SemaphoreType.DMA,
            pltpu.SemaphoreType.DMA,
            pltpu.SemaphoreType.DMA,
            pltpu.SemaphoreType.DMA,
            pltpu.SemaphoreType.DMA,
        ],
    )


@functools.cache
def _sc_head():
    return pl.kernel(
        _sc_head_body,
        mesh=plsc.VectorSubcoreMesh(**_SC_MESH),
        out_type=(),
        scratch_types=[
            pltpu.VMEM((SPANS_PER_W,), jnp.int32),
            pltpu.VMEM((SPANS_PER_W,), jnp.int32),
            pltpu.VMEM((CHUNK_B, DC), jnp.float32),
            pltpu.VMEM((CHUNK_B, DC), jnp.float32),
            pltpu.VMEM((CHUNK_B, DC), jnp.float32),
            pltpu.VMEM((CHUNK_B, DC), jnp.float32),
            pltpu.VMEM((CHUNK_B, D_HEAD), jnp.float32),
            pltpu.VMEM((CHUNK_B, D_HEAD), jnp.float32),
            pltpu.VMEM((CHUNK_B * L,), jnp.float32),
            pltpu.SemaphoreType.DMA,
            pltpu.SemaphoreType.DMA,
            pltpu.SemaphoreType.DMA,
            pltpu.SemaphoreType.DMA,
        ],
    )


def kernel(head_emb, text_lens, context_outputs, span_starts, span_ends,
           is_training, span_width_embeddings, attn_W, attn_b):
    del text_lens, is_training
    starts = span_starts.astype(jnp.int32)
    ends = span_ends.astype(jnp.int32)

    out_ref = jax.new_ref(jax.lax.empty((N_SPANS, D_OUT), jnp.float32))
    _sc_ctx()(context_outputs, span_width_embeddings, starts, ends, out_ref)
    cz = _prefix_table(context_outputs, head_emb, attn_W,
                       attn_b.reshape(1, 1).astype(jnp.float32))
    _sc_head()(cz, starts, ends, out_ref)
    return jax.freeze(out_ref)


# speedup vs baseline: 1.0232x; 1.0187x over previous
import functools

import jax
import jax.numpy as jnp
from jax import lax
from jax.experimental import pallas as pl
from jax.experimental.pallas import tpu as pltpu
from jax.experimental.pallas import tpu_sc as plsc

T = 4096
N_SPANS = 4096
D_HEAD = 1024
D_CTX = 1024
F = 128
ZCOLS = 128
DC = D_HEAD + ZCOLS
D_OUT = D_CTX + D_CTX + F + D_HEAD

BLK = 256
L = 16
NC, NS = 2, 16
NW = NC * NS
SPANS_PER_W = N_SPANS // NW
CHUNK = 8
N_CHUNKS = SPANS_PER_W // CHUNK


def _prefix_body(ctx_ref, head_ref, w_ref, b_ref, out_ref, carry_ref):
    i = pl.program_id(0)

    @pl.when(i == 0)
    def _():
        carry_ref[...] = jnp.zeros_like(carry_ref)

    ctx = ctx_ref[...]
    head = head_ref[...]
    w = w_ref[...]
    b = b_ref[0, 0]
    s = jnp.sum(ctx * w, axis=1, keepdims=True) + b
    e = jnp.exp(s)
    gfull = jnp.concatenate(
        [e * head, jnp.broadcast_to(e, (BLK, ZCOLS))], axis=1)
    r = lax.broadcasted_iota(jnp.int32, (BLK, BLK), 0)
    c = lax.broadcasted_iota(jnp.int32, (BLK, BLK), 1)
    strict_l = (r > c).astype(jnp.float32)
    ex = jnp.dot(strict_l, gfull, preferred_element_type=jnp.float32)
    out_ref[...] = ex + carry_ref[...]
    carry_ref[...] = carry_ref[...] + jnp.sum(gfull, axis=0, keepdims=True)


def _prefix_table(context_outputs, head_emb, attn_w, attn_b):
    return pl.pallas_call(
        _prefix_body,
        grid=(T // BLK,),
        in_specs=[
            pl.BlockSpec((BLK, D_CTX), lambda i: (i, 0)),
            pl.BlockSpec((BLK, D_HEAD), lambda i: (i, 0)),
            pl.BlockSpec((1, D_CTX), lambda i: (0, 0)),
            pl.BlockSpec((1, 1), lambda i: (0, 0)),
        ],
        out_specs=pl.BlockSpec((BLK, DC), lambda i: (i, 0)),
        out_shape=jax.ShapeDtypeStruct((T, DC), jnp.float32),
        scratch_shapes=[pltpu.VMEM((1, DC), jnp.float32)],
    )(context_outputs, head_emb, attn_w, attn_b)


def _worker_base():
    wid = lax.axis_index("s") * NC + lax.axis_index("c")
    return wid * SPANS_PER_W


def _sc_ctx_body(ctx_hbm, swe_hbm, st_hbm, en_hbm, out_hbm,
                 st_all, en_all, wi_all, we_all,
                 cxs0, cxs1, cxe0, cxe1, gsem0, gsem1, osem0, osem1, wsem):
    cxs = (cxs0, cxs1)
    cxe = (cxe0, cxe1)
    gsem = (gsem0, gsem1)
    osem = (osem0, osem1)
    base = _worker_base()

    pltpu.sync_copy(st_hbm.at[pl.ds(base, SPANS_PER_W)], st_all)
    pltpu.sync_copy(en_hbm.at[pl.ds(base, SPANS_PER_W)], en_all)

    @plsc.parallel_loop(0, SPANS_PER_W, step=L)
    def _(t):
        sl = pl.ds(t, L)
        wi_all[sl] = en_all[sl] - st_all[sl]

    pltpu.async_copy(swe_hbm.at[wi_all], we_all, gsem0).wait()

    def fire_g(c, b):
        sl = pl.ds(c * CHUNK_A, CHUNK_A)
        pltpu.async_copy(ctx_hbm.at[st_all.at[sl]], cxs[b], gsem[b])
        pltpu.async_copy(ctx_hbm.at[en_all.at[sl]], cxe[b], gsem[b])

    def wait_g(b):
        pltpu.make_async_copy(ctx_hbm.at[pl.ds(0, CHUNK_A)], cxs[b],
                              gsem[b]).wait()
        pltpu.make_async_copy(ctx_hbm.at[pl.ds(0, CHUNK_A)], cxe[b],
                              gsem[b]).wait()

    def fire_out(c, b):
        rows = pl.ds(base + c * CHUNK_A, CHUNK_A)
        pltpu.async_copy(cxs[b], out_hbm.at[rows, pl.ds(0, D_CTX)], osem[b])
        pltpu.async_copy(cxe[b], out_hbm.at[rows, pl.ds(D_CTX, D_CTX)],
                         osem[b])

    def wait_out(b):
        rows = pl.ds(base, CHUNK_A)
        pltpu.make_async_copy(cxs[b], out_hbm.at[rows, pl.ds(0, D_CTX)],
                              osem[b]).wait()
        pltpu.make_async_copy(cxe[b], out_hbm.at[rows, pl.ds(D_CTX, D_CTX)],
                              osem[b]).wait()

    we_out = out_hbm.at[pl.ds(base, SPANS_PER_W), pl.ds(2 * D_CTX, F)]
    pltpu.async_copy(we_all, we_out, wsem)
    fire_g(0, 0)

    def outer(k, _):
        for b in range(2):
            c = 2 * k + b
            wait_g(b)
            fire_out(c, b)

            @pl.when(c < N_CHUNKS_A - 1)
            def _():
                @pl.when(c > 0)
                def _():
                    wait_out(1 - b)

                fire_g(c + 1, 1 - b)

        return 0

    lax.fori_loop(0, N_CHUNKS_A // 2, outer, 0)
    wait_out(0)
    wait_out(1)
    pltpu.make_async_copy(we_all, we_out, wsem).wait()


def _sc_head_body(cz_hbm, st_hbm, en_hbm, out_hbm,
                  st_all, enp_all, czs0, czs1, czs2, czs3,
                  cze0, cze1, cze2, cze3, sh0, sh1, sh2, sh3,
                  rcp_buf, gsem0, gsem1, gsem2, gsem3,
                  osem0, osem1, osem2, osem3):
    czs = (czs0, czs1, czs2, czs3)
    cze = (cze0, cze1, cze2, cze3)
    sh = (sh0, sh1, sh2, sh3)
    gsem = (gsem0, gsem1, gsem2, gsem3)
    osem = (osem0, osem1, osem2, osem3)
    base = _worker_base()

    pltpu.sync_copy(st_hbm.at[pl.ds(base, SPANS_PER_W)], st_all)
    pltpu.sync_copy(en_hbm.at[pl.ds(base, SPANS_PER_W)], enp_all)

    @plsc.parallel_loop(0, SPANS_PER_W, step=L)
    def _(t):
        sl = pl.ds(t, L)
        enp_all[sl] = enp_all[sl] + 1

    def fire_g(c, b):
        sl = pl.ds(c * CHUNK_B, CHUNK_B)
        pltpu.async_copy(cz_hbm.at[st_all.at[sl]], czs[b], gsem[b])
        pltpu.async_copy(cz_hbm.at[enp_all.at[sl]], cze[b], gsem[b])

    def wait_g(b):
        pltpu.make_async_copy(cz_hbm.at[pl.ds(0, CHUNK_B)], czs[b],
                              gsem[b]).wait()
        pltpu.make_async_copy(cz_hbm.at[pl.ds(0, CHUNK_B)], cze[b],
                              gsem[b]).wait()

    def fire_out(c, b):
        rows = pl.ds(base + c * CHUNK_B, CHUNK_B)
        pltpu.async_copy(sh[b], out_hbm.at[rows, pl.ds(2 * D_CTX + F, D_HEAD)],
                         osem[b])

    def wait_out(b):
        rows = pl.ds(base, CHUNK_B)
        pltpu.make_async_copy(sh[b], out_hbm.at[rows, pl.ds(2 * D_CTX + F,
                                                            D_HEAD)],
                              osem[b]).wait()

    def compute(b):
        @plsc.parallel_loop(0, CHUNK_B)
        def _(i):
            den = cze[b][i, pl.ds(D_HEAD, L)] - czs[b][i, pl.ds(D_HEAD, L)]
            rcp_buf[pl.ds(i * L, L)] = 1.0 / den

        def span_body(i, _):
            rcp = rcp_buf[pl.ds(i * L, L)]

            @plsc.parallel_loop(0, D_HEAD, step=L, unroll=8)
            def _(t):
                vsl = pl.ds(t, L)
                sh[b][i, vsl] = (cze[b][i, vsl] - czs[b][i, vsl]) * rcp

            return 0

        lax.fori_loop(0, CHUNK_B, span_body, 0)

    fire_g(0, 0)
    fire_g(1, 1)
    fire_g(2, 2)

    def outer(k, _):
        for b in range(NBUF):
            c = NBUF * k + b
            nb = (b + 3) % NBUF
            wait_g(b)

            @pl.when(c + 3 < N_CHUNKS_B)
            def _():
                fire_g(c + 3, nb)

            @pl.when(c > NBUF - 1)
            def _():
                wait_out(b)

            compute(b)
            fire_out(c, b)

        return 0

    lax.fori_loop(0, N_CHUNKS_B // NBUF, outer, 0)
    for b in range(NBUF):
        wait_out(b)


CHUNK_A = 16
N_CHUNKS_A = SPANS_PER_W // CHUNK_A
CHUNK_B = 8
N_CHUNKS_B = SPANS_PER_W // CHUNK_B
NBUF = 4

_SC_MESH = dict(core_axis_name="c", subcore_axis_name="s")


@functools.cache
def _sc_ctx():
    return pl.kernel(
        _sc_ctx_body,
        mesh=plsc.VectorSubcoreMesh(**_SC_MESH),
        out_type=(),
        scratch_types=[
            pltpu.VMEM((SPANS_PER_W,), jnp.int32),
            pltpu.VMEM((SPANS_PER_W,), jnp.int32),
            pltpu.VMEM((SPANS_PER_W,), jnp.int32),
            pltpu.VMEM((SPANS_PER_W, F), jnp.float32),
            pltpu.VMEM((CHUNK_A, D_CTX), jnp.float32),
            pltpu.VMEM((CHUNK_A, D_CTX), jnp.float32),
            pltpu.VMEM((CHUNK_A, D_CTX), jnp.float32),
            pltpu.VMEM((CHUNK_A, D_CTX), jnp.float32),
            pltpu.SemaphoreType.DMA,
            pltpu.SemaphoreType.DMA,
            pltpu.SemaphoreType.DMA,
            pltpu.SemaphoreType.DMA,
            pltpu.SemaphoreType.DMA,
        ],
    )


@functools.cache
def _sc_head():
    return pl.kernel(
        _sc_head_body,
        mesh=plsc.VectorSubcoreMesh(**_SC_MESH),
        out_type=(),
        scratch_types=(
            [pltpu.VMEM((SPANS_PER_W,), jnp.int32)] * 2
            + [pltpu.VMEM((CHUNK_B, DC), jnp.float32)] * (2 * NBUF)
            + [pltpu.VMEM((CHUNK_B, D_HEAD), jnp.float32)] * NBUF
            + [pltpu.VMEM((CHUNK_B * L,), jnp.float32)]
            + [pltpu.SemaphoreType.DMA] * (2 * NBUF)
        ),
    )


def kernel(head_emb, text_lens, context_outputs, span_starts, span_ends,
           is_training, span_width_embeddings, attn_W, attn_b):
    del text_lens, is_training
    starts = span_starts.astype(jnp.int32)
    ends = span_ends.astype(jnp.int32)

    out_ref = jax.new_ref(jax.lax.empty((N_SPANS, D_OUT), jnp.float32))
    _sc_ctx()(context_outputs, span_width_embeddings, starts, ends, out_ref)
    cz = _prefix_table(context_outputs, head_emb, attn_W,
                       attn_b.reshape(1, 1).astype(jnp.float32))
    _sc_head()(cz, starts, ends, out_ref)
    return jax.freeze(out_ref)


# speedup vs baseline: 1.0355x; 1.0121x over previous
import functools

import jax
import jax.numpy as jnp
from jax import lax
from jax.experimental import pallas as pl
from jax.experimental.pallas import tpu as pltpu
from jax.experimental.pallas import tpu_sc as plsc

T = 4096
N_SPANS = 4096
D_HEAD = 1024
D_CTX = 1024
F = 128
ZCOLS = 128
DC = D_HEAD + ZCOLS
D_OUT = D_CTX + D_CTX + F + D_HEAD

BLK = 256
L = 16
NC, NS = 2, 16
NW = NC * NS
SPANS_PER_W = N_SPANS // NW
CHUNK = 8
N_CHUNKS = SPANS_PER_W // CHUNK


def _prefix_body(ctx_ref, head_ref, w_ref, b_ref, out_ref, carry_ref):
    i = pl.program_id(0)

    @pl.when(i == 0)
    def _():
        carry_ref[...] = jnp.zeros_like(carry_ref)

    ctx = ctx_ref[...]
    head = head_ref[...]
    w = w_ref[...]
    b = b_ref[0, 0]
    s = jnp.sum(ctx * w, axis=1, keepdims=True) + b
    e = jnp.exp(s)
    gfull = jnp.concatenate(
        [e * head, jnp.broadcast_to(e, (BLK, ZCOLS))], axis=1)
    r = lax.broadcasted_iota(jnp.int32, (BLK, BLK), 0)
    c = lax.broadcasted_iota(jnp.int32, (BLK, BLK), 1)
    strict_l = (r > c).astype(jnp.float32)
    ex = jnp.dot(strict_l, gfull, preferred_element_type=jnp.float32)
    out_ref[...] = ex + carry_ref[...]
    carry_ref[...] = carry_ref[...] + jnp.sum(gfull, axis=0, keepdims=True)


def _prefix_table(context_outputs, head_emb, attn_w, attn_b):
    return pl.pallas_call(
        _prefix_body,
        grid=(T // BLK,),
        in_specs=[
            pl.BlockSpec((BLK, D_CTX), lambda i: (i, 0)),
            pl.BlockSpec((BLK, D_HEAD), lambda i: (i, 0)),
            pl.BlockSpec((1, D_CTX), lambda i: (0, 0)),
            pl.BlockSpec((1, 1), lambda i: (0, 0)),
        ],
        out_specs=pl.BlockSpec((BLK, DC), lambda i: (i, 0)),
        out_shape=jax.ShapeDtypeStruct((T, DC), jnp.float32),
        scratch_shapes=[pltpu.VMEM((1, DC), jnp.float32)],
    )(context_outputs, head_emb, attn_w, attn_b)


def _worker_base():
    wid = lax.axis_index("s") * NC + lax.axis_index("c")
    return wid * SPANS_PER_W


def _sc_ctx_body(ctx_hbm, swe_hbm, st_hbm, en_hbm, out_hbm,
                 st_all, en_all, wi_all, we_all,
                 cxs0, cxs1, cxs2, cxs3, cxe0, cxe1, cxe2, cxe3,
                 gsem0, gsem1, gsem2, gsem3,
                 osem0, osem1, osem2, osem3, wsem):
    cxs = (cxs0, cxs1, cxs2, cxs3)
    cxe = (cxe0, cxe1, cxe2, cxe3)
    gsem = (gsem0, gsem1, gsem2, gsem3)
    osem = (osem0, osem1, osem2, osem3)
    base = _worker_base()

    pltpu.sync_copy(st_hbm.at[pl.ds(base, SPANS_PER_W)], st_all)
    pltpu.sync_copy(en_hbm.at[pl.ds(base, SPANS_PER_W)], en_all)

    @plsc.parallel_loop(0, SPANS_PER_W, step=L)
    def _(t):
        sl = pl.ds(t, L)
        wi_all[sl] = en_all[sl] - st_all[sl]

    pltpu.async_copy(swe_hbm.at[wi_all], we_all, gsem0).wait()

    def fire_g(c, b):
        sl = pl.ds(c * CHUNK_A, CHUNK_A)
        pltpu.async_copy(ctx_hbm.at[st_all.at[sl]], cxs[b], gsem[b])
        pltpu.async_copy(ctx_hbm.at[en_all.at[sl]], cxe[b], gsem[b])

    def wait_g(b):
        pltpu.make_async_copy(ctx_hbm.at[pl.ds(0, CHUNK_A)], cxs[b],
                              gsem[b]).wait()
        pltpu.make_async_copy(ctx_hbm.at[pl.ds(0, CHUNK_A)], cxe[b],
                              gsem[b]).wait()

    def fire_out(c, b):
        rows = pl.ds(base + c * CHUNK_A, CHUNK_A)
        pltpu.async_copy(cxs[b], out_hbm.at[rows, pl.ds(0, D_CTX)], osem[b])
        pltpu.async_copy(cxe[b], out_hbm.at[rows, pl.ds(D_CTX, D_CTX)],
                         osem[b])

    def wait_out(b):
        rows = pl.ds(base, CHUNK_A)
        pltpu.make_async_copy(cxs[b], out_hbm.at[rows, pl.ds(0, D_CTX)],
                              osem[b]).wait()
        pltpu.make_async_copy(cxe[b], out_hbm.at[rows, pl.ds(D_CTX, D_CTX)],
                              osem[b]).wait()

    we_out = out_hbm.at[pl.ds(base, SPANS_PER_W), pl.ds(2 * D_CTX, F)]
    pltpu.async_copy(we_all, we_out, wsem)
    fire_g(0, 0)
    fire_g(1, 1)
    fire_g(2, 2)

    def outer(k, _):
        for b in range(NBUF):
            c = NBUF * k + b
            nb = (b + 3) % NBUF
            wait_g(b)
            fire_out(c, b)

            @pl.when(c + 3 < N_CHUNKS_A)
            def _():
                @pl.when(c > 0)
                def _():
                    wait_out(nb)

                fire_g(c + 3, nb)

        return 0

    lax.fori_loop(0, N_CHUNKS_A // NBUF, outer, 0)
    for b in range(NBUF):
        wait_out(b)
    pltpu.make_async_copy(we_all, we_out, wsem).wait()


def _sc_head_body(cz_hbm, st_hbm, en_hbm, out_hbm,
                  st_all, enp_all, czs0, czs1, czs2, czs3,
                  cze0, cze1, cze2, cze3, sh0, sh1, sh2, sh3,
                  rcp_buf, gsem0, gsem1, gsem2, gsem3,
                  osem0, osem1, osem2, osem3):
    czs = (czs0, czs1, czs2, czs3)
    cze = (cze0, cze1, cze2, cze3)
    sh = (sh0, sh1, sh2, sh3)
    gsem = (gsem0, gsem1, gsem2, gsem3)
    osem = (osem0, osem1, osem2, osem3)
    base = _worker_base()

    pltpu.sync_copy(st_hbm.at[pl.ds(base, SPANS_PER_W)], st_all)
    pltpu.sync_copy(en_hbm.at[pl.ds(base, SPANS_PER_W)], enp_all)

    @plsc.parallel_loop(0, SPANS_PER_W, step=L)
    def _(t):
        sl = pl.ds(t, L)
        enp_all[sl] = enp_all[sl] + 1

    def fire_g(c, b):
        sl = pl.ds(c * CHUNK_B, CHUNK_B)
        pltpu.async_copy(cz_hbm.at[st_all.at[sl]], czs[b], gsem[b])
        pltpu.async_copy(cz_hbm.at[enp_all.at[sl]], cze[b], gsem[b])

    def wait_g(b):
        pltpu.make_async_copy(cz_hbm.at[pl.ds(0, CHUNK_B)], czs[b],
                              gsem[b]).wait()
        pltpu.make_async_copy(cz_hbm.at[pl.ds(0, CHUNK_B)], cze[b],
                              gsem[b]).wait()

    def fire_out(c, b):
        rows = pl.ds(base + c * CHUNK_B, CHUNK_B)
        pltpu.async_copy(sh[b], out_hbm.at[rows, pl.ds(2 * D_CTX + F, D_HEAD)],
                         osem[b])

    def wait_out(b):
        rows = pl.ds(base, CHUNK_B)
        pltpu.make_async_copy(sh[b], out_hbm.at[rows, pl.ds(2 * D_CTX + F,
                                                            D_HEAD)],
                              osem[b]).wait()

    def compute(b):
        @plsc.parallel_loop(0, CHUNK_B)
        def _(i):
            den = cze[b][i, pl.ds(D_HEAD, L)] - czs[b][i, pl.ds(D_HEAD, L)]
            rcp_buf[pl.ds(i * L, L)] = 1.0 / den

        def span_body(i, _):
            rcp = rcp_buf[pl.ds(i * L, L)]

            @plsc.parallel_loop(0, D_HEAD, step=L, unroll=8)
            def _(t):
                vsl = pl.ds(t, L)
                sh[b][i, vsl] = (cze[b][i, vsl] - czs[b][i, vsl]) * rcp

            return 0

        lax.fori_loop(0, CHUNK_B, span_body, 0)

    fire_g(0, 0)
    fire_g(1, 1)
    fire_g(2, 2)

    def outer(k, _):
        for b in range(NBUF):
            c = NBUF * k + b
            nb = (b + 3) % NBUF
            wait_g(b)

            @pl.when(c + 3 < N_CHUNKS_B)
            def _():
                fire_g(c + 3, nb)

            @pl.when(c > NBUF - 1)
            def _():
                wait_out(b)

            compute(b)
            fire_out(c, b)

        return 0

    lax.fori_loop(0, N_CHUNKS_B // NBUF, outer, 0)
    for b in range(NBUF):
        wait_out(b)


CHUNK_A = 8
N_CHUNKS_A = SPANS_PER_W // CHUNK_A
CHUNK_B = 8
N_CHUNKS_B = SPANS_PER_W // CHUNK_B
NBUF = 4

_SC_MESH = dict(core_axis_name="c", subcore_axis_name="s")


@functools.cache
def _sc_ctx():
    return pl.kernel(
        _sc_ctx_body,
        mesh=plsc.VectorSubcoreMesh(**_SC_MESH),
        out_type=(),
        scratch_types=(
            [pltpu.VMEM((SPANS_PER_W,), jnp.int32)] * 3
            + [pltpu.VMEM((SPANS_PER_W, F), jnp.float32)]
            + [pltpu.VMEM((CHUNK_A, D_CTX), jnp.float32)] * (2 * NBUF)
            + [pltpu.SemaphoreType.DMA] * (2 * NBUF + 1)
        ),
    )


@functools.cache
def _sc_head():
    return pl.kernel(
        _sc_head_body,
        mesh=plsc.VectorSubcoreMesh(**_SC_MESH),
        out_type=(),
        scratch_types=(
            [pltpu.VMEM((SPANS_PER_W,), jnp.int32)] * 2
            + [pltpu.VMEM((CHUNK_B, DC), jnp.float32)] * (2 * NBUF)
            + [pltpu.VMEM((CHUNK_B, D_HEAD), jnp.float32)] * NBUF
            + [pltpu.VMEM((CHUNK_B * L,), jnp.float32)]
            + [pltpu.SemaphoreType.DMA] * (2 * NBUF)
        ),
    )


def kernel(head_emb, text_lens, context_outputs, span_starts, span_ends,
           is_training, span_width_embeddings, attn_W, attn_b):
    del text_lens, is_training
    starts = span_starts.astype(jnp.int32)
    ends = span_ends.astype(jnp.int32)

    out_ref = jax.new_ref(jax.lax.empty((N_SPANS, D_OUT), jnp.float32))
    _sc_ctx()(context_outputs, span_width_embeddings, starts, ends, out_ref)
    cz = _prefix_table(context_outputs, head_emb, attn_W,
                       attn_b.reshape(1, 1).astype(jnp.float32))
    _sc_head()(cz, starts, ends, out_ref)
    return jax.freeze(out_ref)


# speedup vs baseline: 1.0377x; 1.0021x over previous
import functools

import jax
import jax.numpy as jnp
from jax import lax
from jax.experimental import pallas as pl
from jax.experimental.pallas import tpu as pltpu
from jax.experimental.pallas import tpu_sc as plsc

T = 4096
N_SPANS = 4096
D_HEAD = 1024
D_CTX = 1024
F = 128
ZCOLS = 128
DC = D_HEAD + ZCOLS
D_OUT = D_CTX + D_CTX + F + D_HEAD

BLK = 512
L = 16
NC, NS = 2, 16
NW = NC * NS
SPANS_PER_W = N_SPANS // NW
CHUNK = 8
N_CHUNKS = SPANS_PER_W // CHUNK


def _prefix_body(ctx_ref, head_ref, w_ref, b_ref, out_ref, carry_ref):
    i = pl.program_id(0)

    @pl.when(i == 0)
    def _():
        carry_ref[...] = jnp.zeros_like(carry_ref)

    ctx = ctx_ref[...]
    head = head_ref[...]
    w = w_ref[...]
    b = b_ref[0, 0]
    s = jnp.sum(ctx * w, axis=1, keepdims=True) + b
    e = jnp.exp(s)
    gfull = jnp.concatenate(
        [e * head, jnp.broadcast_to(e, (BLK, ZCOLS))], axis=1)
    r = lax.broadcasted_iota(jnp.int32, (BLK, BLK), 0)
    c = lax.broadcasted_iota(jnp.int32, (BLK, BLK), 1)
    strict_l = (r > c).astype(jnp.float32)
    ex = jnp.dot(strict_l, gfull, preferred_element_type=jnp.float32)
    out_ref[...] = ex + carry_ref[...]
    carry_ref[...] = carry_ref[...] + jnp.sum(gfull, axis=0, keepdims=True)


def _prefix_table(context_outputs, head_emb, attn_w, attn_b):
    return pl.pallas_call(
        _prefix_body,
        grid=(T // BLK,),
        in_specs=[
            pl.BlockSpec((BLK, D_CTX), lambda i: (i, 0)),
            pl.BlockSpec((BLK, D_HEAD), lambda i: (i, 0)),
            pl.BlockSpec((1, D_CTX), lambda i: (0, 0)),
            pl.BlockSpec((1, 1), lambda i: (0, 0)),
        ],
        out_specs=pl.BlockSpec((BLK, DC), lambda i: (i, 0)),
        out_shape=jax.ShapeDtypeStruct((T, DC), jnp.float32),
        scratch_shapes=[pltpu.VMEM((1, DC), jnp.float32)],
    )(context_outputs, head_emb, attn_w, attn_b)


def _worker_base():
    wid = lax.axis_index("s") * NC + lax.axis_index("c")
    return wid * SPANS_PER_W


def _sc_ctx_body(ctx_hbm, swe_hbm, st_hbm, en_hbm, out_hbm,
                 st_all, en_all, wi_all, we_all,
                 cxs0, cxs1, cxs2, cxs3, cxe0, cxe1, cxe2, cxe3,
                 gsem0, gsem1, gsem2, gsem3,
                 osem0, osem1, osem2, osem3, wsem):
    cxs = (cxs0, cxs1, cxs2, cxs3)
    cxe = (cxe0, cxe1, cxe2, cxe3)
    gsem = (gsem0, gsem1, gsem2, gsem3)
    osem = (osem0, osem1, osem2, osem3)
    base = _worker_base()

    pltpu.sync_copy(st_hbm.at[pl.ds(base, SPANS_PER_W)], st_all)
    pltpu.sync_copy(en_hbm.at[pl.ds(base, SPANS_PER_W)], en_all)

    @plsc.parallel_loop(0, SPANS_PER_W, step=L)
    def _(t):
        sl = pl.ds(t, L)
        wi_all[sl] = en_all[sl] - st_all[sl]

    pltpu.async_copy(swe_hbm.at[wi_all], we_all, gsem0).wait()

    def fire_g(c, b):
        sl = pl.ds(c * CHUNK_A, CHUNK_A)
        pltpu.async_copy(ctx_hbm.at[st_all.at[sl]], cxs[b], gsem[b])
        pltpu.async_copy(ctx_hbm.at[en_all.at[sl]], cxe[b], gsem[b])

    def wait_g(b):
        pltpu.make_async_copy(ctx_hbm.at[pl.ds(0, CHUNK_A)], cxs[b],
                              gsem[b]).wait()
        pltpu.make_async_copy(ctx_hbm.at[pl.ds(0, CHUNK_A)], cxe[b],
                              gsem[b]).wait()

    def fire_out(c, b):
        rows = pl.ds(base + c * CHUNK_A, CHUNK_A)
        pltpu.async_copy(cxs[b], out_hbm.at[rows, pl.ds(0, D_CTX)], osem[b])
        pltpu.async_copy(cxe[b], out_hbm.at[rows, pl.ds(D_CTX, D_CTX)],
                         osem[b])

    def wait_out(b):
        rows = pl.ds(base, CHUNK_A)
        pltpu.make_async_copy(cxs[b], out_hbm.at[rows, pl.ds(0, D_CTX)],
                              osem[b]).wait()
        pltpu.make_async_copy(cxe[b], out_hbm.at[rows, pl.ds(D_CTX, D_CTX)],
                              osem[b]).wait()

    we_out = out_hbm.at[pl.ds(base, SPANS_PER_W), pl.ds(2 * D_CTX, F)]
    pltpu.async_copy(we_all, we_out, wsem)
    fire_g(0, 0)
    fire_g(1, 1)
    fire_g(2, 2)

    def outer(k, _):
        for b in range(NBUF):
            c = NBUF * k + b
            nb = (b + 3) % NBUF
            wait_g(b)
            fire_out(c, b)

            @pl.when(c + 3 < N_CHUNKS_A)
            def _():
                @pl.when(c > 0)
                def _():
                    wait_out(nb)

                fire_g(c + 3, nb)

        return 0

    lax.fori_loop(0, N_CHUNKS_A // NBUF, outer, 0)
    for b in range(NBUF):
        wait_out(b)
    pltpu.make_async_copy(we_all, we_out, wsem).wait()


def _sc_head_body(cz_hbm, st_hbm, en_hbm, out_hbm,
                  st_all, enp_all, czs0, czs1, czs2, czs3,
                  cze0, cze1, cze2, cze3, sh0, sh1, sh2, sh3,
                  rcp_buf, gsem0, gsem1, gsem2, gsem3,
                  osem0, osem1, osem2, osem3):
    czs = (czs0, czs1, czs2, czs3)
    cze = (cze0, cze1, cze2, cze3)
    sh = (sh0, sh1, sh2, sh3)
    gsem = (gsem0, gsem1, gsem2, gsem3)
    osem = (osem0, osem1, osem2, osem3)
    base = _worker_base()

    pltpu.sync_copy(st_hbm.at[pl.ds(base, SPANS_PER_W)], st_all)
    pltpu.sync_copy(en_hbm.at[pl.ds(base, SPANS_PER_W)], enp_all)

    @plsc.parallel_loop(0, SPANS_PER_W, step=L)
    def _(t):
        sl = pl.ds(t, L)
        enp_all[sl] = enp_all[sl] + 1

    def fire_g(c, b):
        sl = pl.ds(c * CHUNK_B, CHUNK_B)
        pltpu.async_copy(cz_hbm.at[st_all.at[sl]], czs[b], gsem[b])
        pltpu.async_copy(cz_hbm.at[enp_all.at[sl]], cze[b], gsem[b])

    def wait_g(b):
        pltpu.make_async_copy(cz_hbm.at[pl.ds(0, CHUNK_B)], czs[b],
                              gsem[b]).wait()
        pltpu.make_async_copy(cz_hbm.at[pl.ds(0, CHUNK_B)], cze[b],
                              gsem[b]).wait()

    def fire_out(c, b):
        rows = pl.ds(base + c * CHUNK_B, CHUNK_B)
        pltpu.async_copy(sh[b], out_hbm.at[rows, pl.ds(2 * D_CTX + F, D_HEAD)],
                         osem[b])

    def wait_out(b):
        rows = pl.ds(base, CHUNK_B)
        pltpu.make_async_copy(sh[b], out_hbm.at[rows, pl.ds(2 * D_CTX + F,
                                                            D_HEAD)],
                              osem[b]).wait()

    def compute(b):
        @plsc.parallel_loop(0, CHUNK_B)
        def _(i):
            den = cze[b][i, pl.ds(D_HEAD, L)] - czs[b][i, pl.ds(D_HEAD, L)]
            rcp_buf[pl.ds(i * L, L)] = 1.0 / den

        def span_body(i, _):
            rcp = rcp_buf[pl.ds(i * L, L)]

            @plsc.parallel_loop(0, D_HEAD, step=L, unroll=8)
            def _(t):
                vsl = pl.ds(t, L)
                sh[b][i, vsl] = (cze[b][i, vsl] - czs[b][i, vsl]) * rcp

            return 0

        lax.fori_loop(0, CHUNK_B, span_body, 0)

    fire_g(0, 0)
    fire_g(1, 1)
    fire_g(2, 2)

    def outer(k, _):
        for b in range(NBUF):
            c = NBUF * k + b
            nb = (b + 3) % NBUF
            wait_g(b)

            @pl.when(c + 3 < N_CHUNKS_B)
            def _():
                fire_g(c + 3, nb)

            @pl.when(c > NBUF - 1)
            def _():
                wait_out(b)

            compute(b)
            fire_out(c, b)

        return 0

    lax.fori_loop(0, N_CHUNKS_B // NBUF, outer, 0)
    for b in range(NBUF):
        wait_out(b)


CHUNK_A = 8
N_CHUNKS_A = SPANS_PER_W // CHUNK_A
CHUNK_B = 8
N_CHUNKS_B = SPANS_PER_W // CHUNK_B
NBUF = 4

_SC_MESH = dict(core_axis_name="c", subcore_axis_name="s")


@functools.cache
def _sc_ctx():
    return pl.kernel(
        _sc_ctx_body,
        mesh=plsc.VectorSubcoreMesh(**_SC_MESH),
        out_type=(),
        scratch_types=(
            [pltpu.VMEM((SPANS_PER_W,), jnp.int32)] * 3
            + [pltpu.VMEM((SPANS_PER_W, F), jnp.float32)]
            + [pltpu.VMEM((CHUNK_A, D_CTX), jnp.float32)] * (2 * NBUF)
            + [pltpu.SemaphoreType.DMA] * (2 * NBUF + 1)
        ),
    )


@functools.cache
def _sc_head():
    return pl.kernel(
        _sc_head_body,
        mesh=plsc.VectorSubcoreMesh(**_SC_MESH),
        out_type=(),
        scratch_types=(
            [pltpu.VMEM((SPANS_PER_W,), jnp.int32)] * 2
            + [pltpu.VMEM((CHUNK_B, DC), jnp.float32)] * (2 * NBUF)
            + [pltpu.VMEM((CHUNK_B, D_HEAD), jnp.float32)] * NBUF
            + [pltpu.VMEM((CHUNK_B * L,), jnp.float32)]
            + [pltpu.SemaphoreType.DMA] * (2 * NBUF)
        ),
    )


def kernel(head_emb, text_lens, context_outputs, span_starts, span_ends,
           is_training, span_width_embeddings, attn_W, attn_b):
    del text_lens, is_training
    starts = span_starts.astype(jnp.int32)
    ends = span_ends.astype(jnp.int32)

    out_ref = jax.new_ref(jax.lax.empty((N_SPANS, D_OUT), jnp.float32))
    _sc_ctx()(context_outputs, span_width_embeddings, starts, ends, out_ref)
    cz = _prefix_table(context_outputs, head_emb, attn_W,
                       attn_b.reshape(1, 1).astype(jnp.float32))
    _sc_head()(cz, starts, ends, out_ref)
    return jax.freeze(out_ref)


# speedup vs baseline: 1.0378x; 1.0001x over previous
import functools

import jax
import jax.numpy as jnp
from jax import lax
from jax.experimental import pallas as pl
from jax.experimental.pallas import tpu as pltpu
from jax.experimental.pallas import tpu_sc as plsc

T = 4096
N_SPANS = 4096
D_HEAD = 1024
D_CTX = 1024
F = 128
ZCOLS = 128
DC = D_HEAD + ZCOLS
D_OUT = D_CTX + D_CTX + F + D_HEAD

BLK = 512
L = 16
NC, NS = 2, 16
NW = NC * NS
SPANS_PER_W = N_SPANS // NW


def _prefix_body(ctx_ref, head_ref, w_ref, b_ref, out_ref, carry_ref):
    i = pl.program_id(0)

    @pl.when(i == 0)
    def _():
        carry_ref[...] = jnp.zeros_like(carry_ref)

    ctx = ctx_ref[...]
    head = head_ref[...]
    w = w_ref[...]
    b = b_ref[0, 0]
    s = jnp.sum(ctx * w, axis=1, keepdims=True) + b
    e = jnp.exp(s)
    gfull = jnp.concatenate(
        [e * head, jnp.broadcast_to(e, (BLK, ZCOLS))], axis=1)
    r = lax.broadcasted_iota(jnp.int32, (BLK, BLK), 0)
    c = lax.broadcasted_iota(jnp.int32, (BLK, BLK), 1)
    strict_l = (r > c).astype(jnp.float32)
    ex = jnp.dot(strict_l, gfull, preferred_element_type=jnp.float32)
    out_ref[...] = ex + carry_ref[...]
    carry_ref[...] = carry_ref[...] + jnp.sum(gfull, axis=0, keepdims=True)


def _prefix_table(context_outputs, head_emb, attn_w, attn_b):
    return pl.pallas_call(
        _prefix_body,
        grid=(T // BLK,),
        in_specs=[
            pl.BlockSpec((BLK, D_CTX), lambda i: (i, 0)),
            pl.BlockSpec((BLK, D_HEAD), lambda i: (i, 0)),
            pl.BlockSpec((1, D_CTX), lambda i: (0, 0)),
            pl.BlockSpec((1, 1), lambda i: (0, 0)),
        ],
        out_specs=pl.BlockSpec((BLK, DC), lambda i: (i, 0)),
        out_shape=jax.ShapeDtypeStruct((T, DC), jnp.float32),
        scratch_shapes=[pltpu.VMEM((1, DC), jnp.float32)],
    )(context_outputs, head_emb, attn_w, attn_b)


def _worker_base():
    wid = lax.axis_index("s") * NC + lax.axis_index("c")
    return wid * SPANS_PER_W


def _sc_ctx_body(ctx_hbm, swe_hbm, st_hbm, en_hbm, out_hbm,
                 st_all, en_all, wi_all, we_all,
                 cxs0, cxs1, cxs2, cxs3, cxe0, cxe1, cxe2, cxe3,
                 gsem0, gsem1, gsem2, gsem3,
                 osem0, osem1, osem2, osem3, wsem):
    cxs = (cxs0, cxs1, cxs2, cxs3)
    cxe = (cxe0, cxe1, cxe2, cxe3)
    gsem = (gsem0, gsem1, gsem2, gsem3)
    osem = (osem0, osem1, osem2, osem3)
    base = _worker_base()

    pltpu.sync_copy(st_hbm.at[pl.ds(base, SPANS_PER_W)], st_all)
    pltpu.sync_copy(en_hbm.at[pl.ds(base, SPANS_PER_W)], en_all)

    @plsc.parallel_loop(0, SPANS_PER_W, step=L)
    def _(t):
        sl = pl.ds(t, L)
        wi_all[sl] = en_all[sl] - st_all[sl]

    pltpu.async_copy(swe_hbm.at[wi_all], we_all, gsem0).wait()

    def fire_g(c, b):
        sl = pl.ds(c * CHUNK_A, CHUNK_A)
        pltpu.async_copy(ctx_hbm.at[st_all.at[sl]], cxs[b], gsem[b])
        pltpu.async_copy(ctx_hbm.at[en_all.at[sl]], cxe[b], gsem[b])

    def wait_g(b):
        pltpu.make_async_copy(ctx_hbm.at[pl.ds(0, CHUNK_A)], cxs[b],
                              gsem[b]).wait()
        pltpu.make_async_copy(ctx_hbm.at[pl.ds(0, CHUNK_A)], cxe[b],
                              gsem[b]).wait()

    def fire_out(c, b):
        rows = pl.ds(base + c * CHUNK_A, CHUNK_A)
        pltpu.async_copy(cxs[b], out_hbm.at[rows, pl.ds(0, D_CTX)], osem[b])
        pltpu.async_copy(cxe[b], out_hbm.at[rows, pl.ds(D_CTX, D_CTX)],
                         osem[b])

    def wait_out(b):
        rows = pl.ds(base, CHUNK_A)
        pltpu.make_async_copy(cxs[b], out_hbm.at[rows, pl.ds(0, D_CTX)],
                              osem[b]).wait()
        pltpu.make_async_copy(cxe[b], out_hbm.at[rows, pl.ds(D_CTX, D_CTX)],
                              osem[b]).wait()

    we_out = out_hbm.at[pl.ds(base, SPANS_PER_W), pl.ds(2 * D_CTX, F)]
    pltpu.async_copy(we_all, we_out, wsem)
    fire_g(0, 0)
    fire_g(1, 1)
    fire_g(2, 2)

    def outer(k, _):
        for b in range(NBUF):
            c = NBUF * k + b
            nb = (b + 3) % NBUF
            wait_g(b)
            fire_out(c, b)

            @pl.when(c + 3 < N_CHUNKS_A)
            def _():
                @pl.when(c > 0)
                def _():
                    wait_out(nb)

                fire_g(c + 3, nb)

        return 0

    lax.fori_loop(0, N_CHUNKS_A // NBUF, outer, 0)
    for b in range(NBUF):
        wait_out(b)
    pltpu.make_async_copy(we_all, we_out, wsem).wait()


def _sc_head_body(cz_hbm, st_hbm, en_hbm, out_hbm,
                  st_all, enp_all, czs0, czs1, czs2, czs3,
                  cze0, cze1, cze2, cze3, sh0, sh1, sh2, sh3,
                  rcp_buf, gsem0, gsem1, gsem2, gsem3,
                  osem0, osem1, osem2, osem3):
    czs = (czs0, czs1, czs2, czs3)
    cze = (cze0, cze1, cze2, cze3)
    sh = (sh0, sh1, sh2, sh3)
    gsem = (gsem0, gsem1, gsem2, gsem3)
    osem = (osem0, osem1, osem2, osem3)
    base = _worker_base()

    pltpu.sync_copy(st_hbm.at[pl.ds(base, SPANS_PER_W)], st_all)
    pltpu.sync_copy(en_hbm.at[pl.ds(base, SPANS_PER_W)], enp_all)

    @plsc.parallel_loop(0, SPANS_PER_W, step=L)
    def _(t):
        sl = pl.ds(t, L)
        enp_all[sl] = enp_all[sl] + 1

    def fire_g(c, b):
        sl = pl.ds(c * CHUNK_B, CHUNK_B)
        pltpu.async_copy(cz_hbm.at[st_all.at[sl]], czs[b], gsem[b])
        pltpu.async_copy(cz_hbm.at[enp_all.at[sl]], cze[b], gsem[b])

    def wait_g(b):
        pltpu.make_async_copy(cz_hbm.at[pl.ds(0, CHUNK_B)], czs[b],
                              gsem[b]).wait()
        pltpu.make_async_copy(cz_hbm.at[pl.ds(0, CHUNK_B)], cze[b],
                              gsem[b]).wait()

    def fire_out(c, b):
        rows = pl.ds(base + c * CHUNK_B, CHUNK_B)
        pltpu.async_copy(sh[b], out_hbm.at[rows, pl.ds(2 * D_CTX + F, D_HEAD)],
                         osem[b])

    def wait_out(b):
        rows = pl.ds(base, CHUNK_B)
        pltpu.make_async_copy(sh[b], out_hbm.at[rows, pl.ds(2 * D_CTX + F,
                                                            D_HEAD)],
                              osem[b]).wait()

    def compute(b):
        @plsc.parallel_loop(0, CHUNK_B)
        def _(i):
            den = cze[b][i, pl.ds(D_HEAD, L)] - czs[b][i, pl.ds(D_HEAD, L)]
            rcp_buf[pl.ds(i * L, L)] = 1.0 / den

        def span_body(i, _):
            rcp = rcp_buf[pl.ds(i * L, L)]

            @plsc.parallel_loop(0, D_HEAD, step=L, unroll=8)
            def _(t):
                vsl = pl.ds(t, L)
                sh[b][i, vsl] = (cze[b][i, vsl] - czs[b][i, vsl]) * rcp

            return 0

        lax.fori_loop(0, CHUNK_B, span_body, 0)

    fire_g(0, 0)
    fire_g(1, 1)
    fire_g(2, 2)

    def outer(k, _):
        for b in range(NBUF):
            c = NBUF * k + b
            nb = (b + 3) % NBUF
            wait_g(b)

            @pl.when(c + 3 < N_CHUNKS_B)
            def _():
                fire_g(c + 3, nb)

            @pl.when(c > NBUF - 1)
            def _():
                wait_out(b)

            compute(b)
            fire_out(c, b)

        return 0

    lax.fori_loop(0, N_CHUNKS_B // NBUF, outer, 0)
    for b in range(NBUF):
        wait_out(b)


CHUNK_A = 8
N_CHUNKS_A = SPANS_PER_W // CHUNK_A
CHUNK_B = 8
N_CHUNKS_B = SPANS_PER_W // CHUNK_B
NBUF = 4

_SC_MESH = dict(core_axis_name="c", subcore_axis_name="s")


@functools.cache
def _sc_ctx():
    return pl.kernel(
        _sc_ctx_body,
        mesh=plsc.VectorSubcoreMesh(**_SC_MESH),
        out_type=(),
        scratch_types=(
            [pltpu.VMEM((SPANS_PER_W,), jnp.int32)] * 3
            + [pltpu.VMEM((SPANS_PER_W, F), jnp.float32)]
            + [pltpu.VMEM((CHUNK_A, D_CTX), jnp.float32)] * (2 * NBUF)
            + [pltpu.SemaphoreType.DMA] * (2 * NBUF + 1)
        ),
    )


@functools.cache
def _sc_head():
    return pl.kernel(
        _sc_head_body,
        mesh=plsc.VectorSubcoreMesh(**_SC_MESH),
        out_type=(),
        scratch_types=(
            [pltpu.VMEM((SPANS_PER_W,), jnp.int32)] * 2
            + [pltpu.VMEM((CHUNK_B, DC), jnp.float32)] * (2 * NBUF)
            + [pltpu.VMEM((CHUNK_B, D_HEAD), jnp.float32)] * NBUF
            + [pltpu.VMEM((CHUNK_B * L,), jnp.float32)]
            + [pltpu.SemaphoreType.DMA] * (2 * NBUF)
        ),
    )


def kernel(head_emb, text_lens, context_outputs, span_starts, span_ends,
           is_training, span_width_embeddings, attn_W, attn_b):
    del text_lens, is_training
    starts = span_starts.astype(jnp.int32)
    ends = span_ends.astype(jnp.int32)

    out_ref = jax.new_ref(jax.lax.empty((N_SPANS, D_OUT), jnp.float32))
    _sc_ctx()(context_outputs, span_width_embeddings, starts, ends, out_ref)
    cz = _prefix_table(context_outputs, head_emb, attn_W,
                       attn_b.reshape(1, 1).astype(jnp.float32))
    _sc_head()(cz, starts, ends, out_ref)
    return jax.freeze(out_ref)
